# Initial kernel scaffold; baseline (speedup 1.0000x reference)
import jax
import jax.numpy as jnp
from jax.experimental import pallas as pl


def kernel(logits, tour_edges, dist):
    raise NotImplementedError("write your pallas kernel here")


# speedup vs baseline: 32.2048x; 32.2048x over previous
import functools

import jax
import jax.numpy as jnp
from jax import lax
from jax.experimental import pallas as pl
from jax.experimental.pallas import tpu as pltpu
from jax.experimental.pallas import tpu_sc as plsc

_B = 4
_N = 256
_BIG_MULT = 10.0
_COEF = 0.15 * 1 / 10000

_L = 16
_NC = 2
_NCHUNK = _N // _L
_SENT = 3.0e38
_BIGKEY = 2 ** 30


def _build_kernel(logits_ref, ei_ref, ej_ref, dist_ref, w_ref, l1_ref):
    dist = dist_ref[...]
    big = _BIG_MULT * jnp.max(dist)
    iota_col = lax.broadcasted_iota(jnp.int32, (_N, _N), 0)
    iota_row = lax.broadcasted_iota(jnp.int32, (_N, _N), 1)
    diag = iota_col == iota_row
    l1_acc = jnp.float32(0.0)
    for b in range(_B):
        db = dist[b]
        dbt = db.T
        ei = ei_ref[b].reshape(1, _N)
        ej = ej_ref[b].reshape(1, _N)
        ut = (iota_col == ei).astype(jnp.float32)
        vt = (iota_col == ej).astype(jnp.float32)
        cdims = (((1,), (1,)), ((), ()))
        m = lax.dot_general(ut, vt, cdims, preferred_element_type=jnp.float32)
        mt = lax.dot_general(vt, ut, cdims, preferred_element_type=jnp.float32)
        wa = jnp.where(mt > 0.5, dbt, jnp.where(m > 0.5, db, big))
        wa = jnp.where(diag, 0.0, wa)
        l0 = logits_ref[b, 0]
        l1v = logits_ref[b, 1]
        p = 1.0 / (1.0 + jnp.exp(l0 - l1v))
        wb = (1.0 - p) * big + p * db
        w_ref[b] = wa
        w_ref[_B + b] = jnp.minimum(wa, wb)
        l1_acc = l1_acc + jnp.sum(jnp.abs(wa - wb))
    l1_ref[0, 0] = l1_acc / jnp.float32(_B * _N * _N)


def _lane_min(x):
    lane = lax.broadcasted_iota(jnp.int32, (_L,), 0)
    for sh in (8, 4, 2, 1):
        idx = jnp.bitwise_xor(lane, sh)
        x = jnp.minimum(x, x.at[idx].get(mode="promise_in_bounds"))
    return x


def _sc_prim_body(w_hbm, out_hbm, w_v, md_v, bu_v, out_v):
    c = lax.axis_index("c")
    s = lax.axis_index("s")
    wid = s * _NC + c

    @pl.when(wid < 2 * _B)
    def _():
        pltpu.sync_copy(w_hbm.at[wid], w_v)
        lane = lax.broadcasted_iota(jnp.int32, (_L,), 0)
        for ck in range(_NCHUNK):
            row = w_v[pl.ds(ck * _L, _L)]
            if ck == 0:
                row = jnp.where(lane == 0, _SENT, row)
            md_v[pl.ds(ck * _L, _L)] = row
            bu_v[pl.ds(ck * _L, _L)] = jnp.zeros((_L,), jnp.int32)

        def step(_, total):
            acc = md_v[pl.ds(0, _L)]
            for ck in range(1, _NCHUNK):
                acc = jnp.minimum(acc, md_v[pl.ds(ck * _L, _L)])
            mvec = _lane_min(acc)
            kacc = jnp.full((_L,), _BIGKEY, jnp.int32)
            for ck in range(_NCHUNK):
                md = md_v[pl.ds(ck * _L, _L)]
                bu = bu_v[pl.ds(ck * _L, _L)]
                key = jnp.where(md == mvec, bu * _N + (lane + ck * _L), _BIGKEY)
                kacc = jnp.minimum(kacc, key)
            vvec = jnp.bitwise_and(_lane_min(kacc), _N - 1)
            for ck in range(_NCHUNK):
                col = lane + ck * _L
                row = plsc.load_gather(w_v, [vvec * _N + col])
                old = md_v[pl.ds(ck * _L, _L)]
                bu = bu_v[pl.ds(ck * _L, _L)]
                lt = row < old
                eq = row == old
                bu_v[pl.ds(ck * _L, _L)] = jnp.where(
                    lt, vvec, jnp.where(eq, jnp.minimum(bu, vvec), bu))
                nmd = jnp.where(lt, row, old)
                dead = (old == _SENT) | (col == vvec)
                md_v[pl.ds(ck * _L, _L)] = jnp.where(dead, _SENT, nmd)
            return total + mvec

        total = lax.fori_loop(0, _N - 1, step, jnp.zeros((_L,), jnp.float32))
        out_v[...] = total
        pltpu.sync_copy(out_v, out_hbm.at[wid])


@functools.cache
def _sc_prim():
    return pl.kernel(
        _sc_prim_body,
        out_type=jax.ShapeDtypeStruct((2 * _B, _L), jnp.float32),
        mesh=plsc.VectorSubcoreMesh(
            core_axis_name="c", subcore_axis_name="s",
            num_cores=_NC, num_subcores=16),
        compiler_params=pltpu.CompilerParams(needs_layout_passes=False),
        scratch_types=[
            pltpu.VMEM((_N * _N,), jnp.float32),
            pltpu.VMEM((_N,), jnp.float32),
            pltpu.VMEM((_N,), jnp.int32),
            pltpu.VMEM((_L,), jnp.float32),
        ],
    )


@jax.jit
def kernel(logits, tour_edges, dist):
    ei = tour_edges[..., 0].astype(jnp.int32)
    ej = tour_edges[..., 1].astype(jnp.int32)
    w, l1 = pl.pallas_call(
        _build_kernel,
        out_shape=(
            jax.ShapeDtypeStruct((2 * _B, _N, _N), jnp.float32),
            jax.ShapeDtypeStruct((1, 1), jnp.float32),
        ),
        in_specs=[
            pl.BlockSpec(memory_space=pltpu.VMEM),
            pl.BlockSpec(memory_space=pltpu.VMEM),
            pl.BlockSpec(memory_space=pltpu.VMEM),
            pl.BlockSpec(memory_space=pltpu.VMEM),
        ],
        out_specs=(
            pl.BlockSpec(memory_space=pltpu.VMEM),
            pl.BlockSpec(memory_space=pltpu.SMEM),
        ),
    )(logits, ei, ej, dist)
    totals = _sc_prim()(w.reshape(2 * _B, _N * _N))[:, 0]
    mst_a = totals[:_B]
    mst_c = totals[_B:]
    topo = jnp.mean(mst_a - mst_c) + 0.001 * l1[0, 0]
    return _COEF * topo


# speedup vs baseline: 37.4963x; 1.1643x over previous
import functools

import jax
import jax.numpy as jnp
from jax import lax
from jax.experimental import pallas as pl
from jax.experimental.pallas import tpu as pltpu
from jax.experimental.pallas import tpu_sc as plsc

_B = 4
_N = 256
_BIG_MULT = 10.0
_COEF = 0.15 * 1 / 10000

_L = 16
_NC = 2
_NCHUNK = _N // _L
_SENT = 3.0e38
_BIGKEY = 2 ** 30


def _build_kernel(logits_ref, ei_ref, ej_ref, dist_ref, w_ref, l1_ref):
    dist = dist_ref[...]
    big = _BIG_MULT * jnp.max(dist)
    iota_col = lax.broadcasted_iota(jnp.int32, (_N, _N), 0)
    iota_row = lax.broadcasted_iota(jnp.int32, (_N, _N), 1)
    diag = iota_col == iota_row
    l1_acc = jnp.float32(0.0)
    for b in range(_B):
        db = dist[b]
        dbt = db.T
        ei = ei_ref[b].reshape(1, _N)
        ej = ej_ref[b].reshape(1, _N)
        ut = (iota_col == ei).astype(jnp.float32)
        vt = (iota_col == ej).astype(jnp.float32)
        cdims = (((1,), (1,)), ((), ()))
        m = lax.dot_general(ut, vt, cdims, preferred_element_type=jnp.float32)
        mt = lax.dot_general(vt, ut, cdims, preferred_element_type=jnp.float32)
        wa = jnp.where(mt > 0.5, dbt, jnp.where(m > 0.5, db, big))
        wa = jnp.where(diag, 0.0, wa)
        l0 = logits_ref[b, 0]
        l1v = logits_ref[b, 1]
        p = 1.0 / (1.0 + jnp.exp(l0 - l1v))
        wb = (1.0 - p) * big + p * db
        w_ref[b] = wa
        w_ref[_B + b] = jnp.minimum(wa, wb)
        l1_acc = l1_acc + jnp.sum(jnp.abs(wa - wb))
    l1_ref[0, 0] = l1_acc / jnp.float32(_B * _N * _N)


def _lane_min(x):
    lane = lax.broadcasted_iota(jnp.int32, (_L,), 0)
    for sh in (8, 4, 2, 1):
        idx = jnp.bitwise_xor(lane, sh)
        x = jnp.minimum(x, x.at[idx].get(mode="promise_in_bounds"))
    return x


def _sc_prim_body(w_hbm, out_hbm, w_v, md_v, out_v):
    c = lax.axis_index("c")
    s = lax.axis_index("s")
    wid = s * _NC + c

    @pl.when(wid < 2 * _B)
    def _():
        pltpu.sync_copy(w_hbm.at[wid], w_v)
        lane = lax.broadcasted_iota(jnp.int32, (_L,), 0)
        acc0 = jnp.full((_L,), _SENT, jnp.float32)
        for ck in range(_NCHUNK):
            row = w_v[pl.ds(ck * _L, _L)]
            if ck == 0:
                row = jnp.where(lane == 0, _SENT, row)
            md_v[pl.ds(ck * _L, _L)] = row
            acc0 = jnp.minimum(acc0, row)

        def step(_, carry):
            total, acc = carry
            mvec = _lane_min(acc)
            kacc = jnp.full((_L,), _BIGKEY, jnp.int32)
            for ck in range(_NCHUNK):
                md = md_v[pl.ds(ck * _L, _L)]
                key = jnp.where(md == mvec, lane + ck * _L, _BIGKEY)
                kacc = jnp.minimum(kacc, key)
            vvec = _lane_min(kacc)
            nacc = jnp.full((_L,), _SENT, jnp.float32)
            for ck in range(_NCHUNK):
                col = lane + ck * _L
                row = plsc.load_gather(w_v, [vvec * _N + col])
                old = md_v[pl.ds(ck * _L, _L)]
                nmd = jnp.minimum(old, row)
                dead = (old == _SENT) | (col == vvec)
                nmd = jnp.where(dead, _SENT, nmd)
                md_v[pl.ds(ck * _L, _L)] = nmd
                nacc = jnp.minimum(nacc, nmd)
            return total + mvec, nacc

        total, _ = lax.fori_loop(
            0, _N - 1, step,
            (jnp.zeros((_L,), jnp.float32), acc0))
        out_v[...] = total
        pltpu.sync_copy(out_v, out_hbm.at[wid])


@functools.cache
def _sc_prim():
    return pl.kernel(
        _sc_prim_body,
        out_type=jax.ShapeDtypeStruct((2 * _B, _L), jnp.float32),
        mesh=plsc.VectorSubcoreMesh(
            core_axis_name="c", subcore_axis_name="s",
            num_cores=_NC, num_subcores=16),
        compiler_params=pltpu.CompilerParams(needs_layout_passes=False),
        scratch_types=[
            pltpu.VMEM((_N * _N,), jnp.float32),
            pltpu.VMEM((_N,), jnp.float32),
            pltpu.VMEM((_L,), jnp.float32),
        ],
    )


@jax.jit
def kernel(logits, tour_edges, dist):
    ei = tour_edges[..., 0].astype(jnp.int32)
    ej = tour_edges[..., 1].astype(jnp.int32)
    w, l1 = pl.pallas_call(
        _build_kernel,
        out_shape=(
            jax.ShapeDtypeStruct((2 * _B, _N, _N), jnp.float32),
            jax.ShapeDtypeStruct((1, 1), jnp.float32),
        ),
        in_specs=[
            pl.BlockSpec(memory_space=pltpu.VMEM),
            pl.BlockSpec(memory_space=pltpu.VMEM),
            pl.BlockSpec(memory_space=pltpu.VMEM),
            pl.BlockSpec(memory_space=pltpu.VMEM),
        ],
        out_specs=(
            pl.BlockSpec(memory_space=pltpu.VMEM),
            pl.BlockSpec(memory_space=pltpu.SMEM),
        ),
    )(logits, ei, ej, dist)
    totals = _sc_prim()(w.reshape(2 * _B, _N * _N))[:, 0]
    mst_a = totals[:_B]
    mst_c = totals[_B:]
    topo = jnp.mean(mst_a - mst_c) + 0.001 * l1[0, 0]
    return _COEF * topo
